# baseline (speedup 1.0000x reference)
import jax
import jax.numpy as jnp
from jax import lax
from jax.experimental import pallas as pl
from jax.experimental.pallas import tpu as pltpu
from jax.experimental.pallas import tpu_sc as plsc

B, F, D = 4096, 26, 64
MAXV = 1000000
NC, NS = 2, 16
NW = NC * NS
RPW = B // NW
BC = 32
NCHUNK = RPW // BC
GPC = BC * F
STREAMS = [128] * 6 + [64]


def _fm_body(xr, xt, bias_t, fm_t, out1, out2, idx_v, idxt_v, rows_v,
             bias_v, bsum_v, stage_v, out1_v, gsem, bsem):
    w = lax.axis_index("s") * NC + lax.axis_index("c")
    for c in range(NCHUNK):
        row0 = w * RPW + c * BC
        g0 = row0 * F
        pltpu.sync_copy(xr.at[pl.ds(g0, GPC)], idx_v)
        pltpu.sync_copy(xt.at[pl.ds(g0, GPC)], idxt_v)
        copies = []
        off = 0
        for n in STREAMS:
            copies.append(pltpu.async_copy(
                fm_t.at[idx_v.at[pl.ds(off, n)]],
                rows_v.at[pl.ds(off, n)], gsem))
            copies.append(pltpu.async_copy(
                bias_t.at[idxt_v.at[pl.ds(off, n)]],
                bias_v.at[pl.ds(off, n)], bsem))
            off += n
        for cp in copies:
            cp.wait()

        for b0 in range(0, BC, 16):
            s = jnp.zeros((16,), jnp.float32)
            for f in range(F):
                s = s + bias_v[pl.ds(f * BC + b0, 16)]
            bsum_v[pl.ds(b0, 16)] = s

        for g in range(BC // 16):
            s16 = bsum_v[pl.ds(g * 16, 16)]

            def body(k, _, g=g, s16=s16):
                b = g * 16 + k
                base = b * F
                s = lax.gather(
                    s16, jnp.full((16, 1), k, jnp.int32),
                    lax.GatherDimensionNumbers(
                        offset_dims=(), collapsed_slice_dims=(0,),
                        start_index_map=(0,)),
                    slice_sizes=(1,),
                    mode=lax.GatherScatterMode.PROMISE_IN_BOUNDS)
                for d in range(4):
                    acc = jnp.zeros((16,), jnp.float32)
                    sq = jnp.zeros((16,), jnp.float32)
                    for f in range(F):
                        v = rows_v[base + f, pl.ds(d * 16, 16)]
                        acc = acc + v
                        sq = sq + v * v
                        stage_v[b, pl.ds(f * D + d * 16, 16)] = v
                    out1_v[b, pl.ds(d * 16, 16)] = s + 0.5 * (acc * acc - sq)
                return 0

            lax.fori_loop(0, 16, body, 0)
        pltpu.sync_copy(stage_v, out2.at[pl.ds(row0, BC)])
        pltpu.sync_copy(out1_v, out1.at[pl.ds(row0, BC)])


@jax.jit
def kernel(x, bias_table, fm_table):
    xr = x.reshape(B * F)
    xt = x.reshape(B // BC, BC, F).transpose(0, 2, 1).reshape(B * F)
    bias_flat = bias_table.reshape(MAXV)
    mesh = plsc.VectorSubcoreMesh(
        core_axis_name="c", subcore_axis_name="s",
        num_cores=NC, num_subcores=NS)
    run = pl.kernel(
        _fm_body,
        out_type=[jax.ShapeDtypeStruct((B, D), jnp.float32),
                  jax.ShapeDtypeStruct((B, F * D), jnp.float32)],
        mesh=mesh,
        scratch_types=[
            pltpu.VMEM((GPC,), jnp.int32),
            pltpu.VMEM((GPC,), jnp.int32),
            pltpu.VMEM((GPC, D), jnp.float32),
            pltpu.VMEM((GPC,), jnp.float32),
            pltpu.VMEM((BC,), jnp.float32),
            pltpu.VMEM((BC, F * D), jnp.float32),
            pltpu.VMEM((BC, D), jnp.float32),
            pltpu.SemaphoreType.DMA,
            pltpu.SemaphoreType.DMA,
        ],
        compiler_params=pltpu.CompilerParams(use_tc_tiling_on_sc=False),
    )
    out1, out2 = run(xr, xt, bias_flat, fm_table)
    return out1, out2


# speedup vs baseline: 1.4210x; 1.4210x over previous
import jax
import jax.numpy as jnp
from jax import lax
from jax.experimental.layout import Format, Layout, with_layout_constraint
from jax.experimental import pallas as pl
from jax.experimental.pallas import tpu as pltpu
from jax.experimental.pallas import tpu_sc as plsc

B, F, D = 4096, 26, 64
MAXV = 1000000
NC, NS = 2, 16
NW = NC * NS
RPW = B // NW
BC = 32
NCHUNK = RPW // BC
GPC = BC * F
STREAMS = [128] * 6 + [64]


def _fm_body(xr, xt, bias_t, fm_t, out1, out2, idx_v, idxt_v, rows_v,
             bias_v, bsum_v, stage_v, out1_v, gsem, bsem):
    w = lax.axis_index("s") * NC + lax.axis_index("c")
    for c in range(NCHUNK):
        row0 = w * RPW + c * BC
        g0 = row0 * F
        pltpu.sync_copy(xr.at[pl.ds(g0, GPC)], idx_v)
        pltpu.sync_copy(xt.at[pl.ds(g0, GPC)], idxt_v)
        copies = []
        off = 0
        for n in STREAMS:
            copies.append(pltpu.async_copy(
                fm_t.at[idx_v.at[pl.ds(off, n)]],
                rows_v.at[pl.ds(off, n)], gsem))
            copies.append(pltpu.async_copy(
                bias_t.at[idxt_v.at[pl.ds(off, n)]],
                bias_v.at[pl.ds(off, n)], bsem))
            off += n
        for cp in copies:
            cp.wait()

        for b0 in range(0, BC, 16):
            s = jnp.zeros((16,), jnp.float32)
            for f in range(F):
                s = s + bias_v[pl.ds(f * BC + b0, 16)]
            bsum_v[pl.ds(b0, 16)] = s

        for g in range(BC // 16):
            s16 = bsum_v[pl.ds(g * 16, 16)]

            def body(k, _, g=g, s16=s16):
                b = g * 16 + k
                base = b * F
                s = lax.gather(
                    s16, jnp.full((16, 1), k, jnp.int32),
                    lax.GatherDimensionNumbers(
                        offset_dims=(), collapsed_slice_dims=(0,),
                        start_index_map=(0,)),
                    slice_sizes=(1,),
                    mode=lax.GatherScatterMode.PROMISE_IN_BOUNDS)
                for d in range(4):
                    acc = jnp.zeros((16,), jnp.float32)
                    sq = jnp.zeros((16,), jnp.float32)
                    for f in range(F):
                        v = rows_v[base + f, pl.ds(d * 16, 16)]
                        acc = acc + v
                        sq = sq + v * v
                        stage_v[b, pl.ds(f * D + d * 16, 16)] = v
                    out1_v[b, pl.ds(d * 16, 16)] = s + 0.5 * (acc * acc - sq)
                return 0

            lax.fori_loop(0, 16, body, 0)
        pltpu.sync_copy(stage_v, out2.at[pl.ds(row0, BC)])
        pltpu.sync_copy(out1_v, out1.at[pl.ds(row0, BC)])


@jax.jit
def kernel(x, bias_table, fm_table):
    xr = x.reshape(B * F)
    xt = x.reshape(B // BC, BC, F).transpose(0, 2, 1).reshape(B * F)
    bias_flat = bias_table.reshape(MAXV)
    fm_table = with_layout_constraint(
        fm_table, Layout(major_to_minor=(0, 1), tiling=((8,),)))
    mesh = plsc.VectorSubcoreMesh(
        core_axis_name="c", subcore_axis_name="s",
        num_cores=NC, num_subcores=NS)
    run = pl.kernel(
        _fm_body,
        out_type=[jax.ShapeDtypeStruct((B, D), jnp.float32),
                  jax.ShapeDtypeStruct((B, F * D), jnp.float32)],
        mesh=mesh,
        scratch_types=[
            pltpu.VMEM((GPC,), jnp.int32),
            pltpu.VMEM((GPC,), jnp.int32),
            pltpu.VMEM((GPC, D), jnp.float32),
            pltpu.VMEM((GPC,), jnp.float32),
            pltpu.VMEM((BC,), jnp.float32),
            pltpu.VMEM((BC, F * D), jnp.float32),
            pltpu.VMEM((BC, D), jnp.float32),
            pltpu.SemaphoreType.DMA,
            pltpu.SemaphoreType.DMA,
        ],
        compiler_params=pltpu.CompilerParams(use_tc_tiling_on_sc=False),
    )
    out1, out2 = run(xr, xt, bias_flat, fm_table)
    return out1, out2
